# baseline (speedup 1.0000x reference)
import functools

import jax
import jax.numpy as jnp
from jax import lax
from jax.experimental import pallas as pl
from jax.experimental.pallas import tpu as pltpu
from jax.experimental.pallas import tpu_sc as plsc

_NUM_NODES = 1000
_N = 4096
_D_SPATIAL = 125
_TOKEN_DIM = 128
_DEG = 31
_MAX_LENGTH = 64
_M = _DEG + 1

_NC = 2
_NS = 16
_NW = _NC * _NS
_TPW = _N // _NW
_L = 16
_NBUF = 4
_BATCH = 1


def _tokenizer_body(data, emb, node1, nbrs, mu1, sig1, tim, out,
                    ids_v, nd_v, nb_v, vals_v, tim_v, mu_v, sig_v,
                    bufs, gsem, osem):
  wid = lax.axis_index("s") * _NC + lax.axis_index("c")
  t0 = wid * _TPW

  pltpu.sync_copy(node1, nd_v.at[pl.ds(0, 1)])
  pltpu.sync_copy(nbrs, nb_v.at[pl.ds(0, _DEG)])
  pltpu.sync_copy(mu1, mu_v.at[pl.ds(0, 1)])
  pltpu.sync_copy(sig1, sig_v.at[pl.ds(0, 1)])
  pltpu.sync_copy(tim.at[pl.ds(t0, _TPW)], tim_v)

  iota = lax.iota(jnp.int32, _L)
  plsc.store_scatter(ids_v, [iota], nd_v[...], mask=iota == 0)
  plsc.store_scatter(ids_v, [iota + 1], nb_v[pl.ds(0, _L)])
  plsc.store_scatter(ids_v, [iota + _L + 1], nb_v[pl.ds(_L, _L)],
                     mask=iota < _L - 1)

  zv = jnp.zeros((_L,), jnp.float32)

  def init_tail(j, carry):
    for b in range(_NBUF):
      for k in range(_BATCH):
        bufs[b, k, j, pl.ds(_TOKEN_DIM - _L, _L)] = zv
    return carry
  lax.fori_loop(0, _M, init_tail, 0)

  idv = [ids_v[pl.ds(g * _L, _L)] for g in range(_M // _L)]
  for j in range(_M):
    idj = idv[j // _L][j % _L]
    pltpu.make_async_copy(data.at[idj, pl.ds(t0, _TPW)],
                          vals_v.at[j], gsem).start()
    for b in range(_NBUF):
      pltpu.make_async_copy(emb.at[idj],
                            bufs.at[b, 0, j, pl.ds(0, _D_SPATIAL)],
                            gsem).start()
  for j in range(_M):
    pltpu.make_async_copy(data.at[0, pl.ds(t0, _TPW)],
                          vals_v.at[j], gsem).wait()
    for b in range(_NBUF):
      pltpu.make_async_copy(emb.at[0],
                            bufs.at[b, 0, j, pl.ds(0, _D_SPATIAL)],
                            gsem).wait()

  mu_r = jnp.full((_L,), mu_v[...][0], jnp.float32)
  inv_s = 1.0 / jnp.full((_L,), sig_v[...][0], jnp.float32)

  def init_zero(i, carry):
    j = _M + i // (_TOKEN_DIM // _L)
    c = (i % (_TOKEN_DIM // _L)) * _L
    for b in range(_NBUF):
      for k in range(_BATCH):
        bufs[b, k, j, pl.ds(c, _L)] = zv
    return carry
  lax.fori_loop(0, (_MAX_LENGTH - _M) * (_TOKEN_DIM // _L), init_zero, 0)

  rows = [lax.iota(jnp.int32, _L) + g * _L for g in range(_M // _L)]
  c125 = jnp.full((_L,), 125, jnp.int32)
  c126 = jnp.full((_L,), 126, jnp.int32)
  c127 = jnp.full((_L,), 127, jnp.int32)
  zc = jnp.zeros((_L,), jnp.int32)
  oc = jnp.full((_L,), 1, jnp.int32)

  def round_body(r, carry):
    for b in range(_NBUF):
      tb = r * _NBUF * _BATCH + b * _BATCH

      @pl.when(r > 0)
      def _wait():
        pltpu.make_async_copy(bufs.at[b], out.at[pl.ds(t0, _BATCH)],
                              osem.at[b]).wait()

      for k in range(_BATCH):
        tt = tb + k
        tts = jnp.full((_L,), tt, jnp.int32)
        t0v = plsc.load_gather(tim_v, [tts, zc])
        t1v = plsc.load_gather(tim_v, [tts, oc])
        for g in range(_M // _L):
          v = plsc.load_gather(vals_v, [rows[g], tts])
          plsc.store_scatter(bufs.at[b, k], [rows[g], c125],
                             (v - mu_r) * inv_s)
          plsc.store_scatter(bufs.at[b, k], [rows[g], c126], t0v)
          plsc.store_scatter(bufs.at[b, k], [rows[g], c127], t1v)

      pltpu.make_async_copy(bufs.at[b], out.at[pl.ds(t0 + tb, _BATCH)],
                            osem.at[b]).start()
    return carry

  lax.fori_loop(0, _TPW // (_NBUF * _BATCH), round_body, 0)

  for b in range(_NBUF):
    pltpu.make_async_copy(bufs.at[b], out.at[pl.ds(t0, _BATCH)],
                          osem.at[b]).wait()


@jax.jit
def _tokenize(data, emb, node1, nbrs, mu1, sig1, tim):
  mesh = plsc.VectorSubcoreMesh(core_axis_name="c", subcore_axis_name="s",
                                num_cores=_NC, num_subcores=_NS)
  f = functools.partial(
      pl.kernel,
      out_type=jax.ShapeDtypeStruct((_N, _MAX_LENGTH, _TOKEN_DIM),
                                    jnp.float32),
      mesh=mesh,
      compiler_params=pltpu.CompilerParams(needs_layout_passes=False),
      scratch_types=[
          pltpu.VMEM((_M,), jnp.int32),
          pltpu.VMEM((_L,), jnp.int32),
          pltpu.VMEM((_M,), jnp.int32),
          pltpu.VMEM((_M, _TPW), jnp.float32),
          pltpu.VMEM((_TPW, 2), jnp.float32),
          pltpu.VMEM((_L,), jnp.float32),
          pltpu.VMEM((_L,), jnp.float32),
          pltpu.VMEM((_NBUF, _BATCH, _MAX_LENGTH, _TOKEN_DIM),
                     jnp.float32),
          pltpu.SemaphoreType.DMA,
          pltpu.SemaphoreType.DMA((_NBUF,)),
      ],
  )(_tokenizer_body)
  return f(data, emb, node1, nbrs, mu1, sig1, tim)


def kernel(data, node, spatial_embedding, temporal_all, neighbors, zt_mu,
           zt_sigma):
  node1 = jnp.asarray(node, jnp.int32).reshape(1)
  return _tokenize(data, spatial_embedding, node1,
                   neighbors.astype(jnp.int32),
                   zt_mu.astype(jnp.float32), zt_sigma.astype(jnp.float32),
                   temporal_all)


# speedup vs baseline: 1.2415x; 1.2415x over previous
import functools

import jax
import jax.numpy as jnp
from jax import lax
from jax.experimental import pallas as pl
from jax.experimental.pallas import tpu as pltpu
from jax.experimental.pallas import tpu_sc as plsc

_NUM_NODES = 1000
_N = 4096
_D_SPATIAL = 125
_TOKEN_DIM = 128
_DEG = 31
_MAX_LENGTH = 64
_M = _DEG + 1

_NC = 2
_NS = 16
_NW = _NC * _NS
_TPW = _N // _NW
_L = 16
_NBUF = 4
_BATCH = 1


def _tokenizer_body(data, emb, node1, nbrs, mu1, sig1, tim, out,
                    ids_v, nd_v, nb_v, emb_v, vals_v, tim_v, mu_v, sig_v,
                    bufs, gsem, osem):
  wid = lax.axis_index("s") * _NC + lax.axis_index("c")
  t0 = wid * _TPW

  pltpu.sync_copy(node1, nd_v.at[pl.ds(0, 1)])
  pltpu.sync_copy(nbrs, nb_v.at[pl.ds(0, _DEG)])
  pltpu.sync_copy(mu1, mu_v.at[pl.ds(0, 1)])
  pltpu.sync_copy(sig1, sig_v.at[pl.ds(0, 1)])
  pltpu.sync_copy(tim.at[pl.ds(t0, _TPW)], tim_v)

  iota = lax.iota(jnp.int32, _L)
  plsc.store_scatter(ids_v, [iota], nd_v[...], mask=iota == 0)
  plsc.store_scatter(ids_v, [iota + 1], nb_v[pl.ds(0, _L)])
  plsc.store_scatter(ids_v, [iota + _L + 1], nb_v[pl.ds(_L, _L)],
                     mask=iota < _L - 1)

  zv = jnp.zeros((_L,), jnp.float32)

  def init_tail(j, carry):
    emb_v[j, pl.ds(_TOKEN_DIM - _L, _L)] = zv
    return carry
  lax.fori_loop(0, _M, init_tail, 0)

  idv = [ids_v[pl.ds(g * _L, _L)] for g in range(_M // _L)]
  for j in range(_M):
    idj = idv[j // _L][j % _L]
    pltpu.make_async_copy(data.at[idj, pl.ds(t0, _TPW)],
                          vals_v.at[j], gsem).start()
    pltpu.make_async_copy(emb.at[idj], emb_v.at[j, pl.ds(0, _D_SPATIAL)],
                          gsem).start()
  for j in range(_M):
    pltpu.make_async_copy(data.at[0, pl.ds(t0, _TPW)],
                          vals_v.at[j], gsem).wait()
    pltpu.make_async_copy(emb.at[0], emb_v.at[j, pl.ds(0, _D_SPATIAL)],
                          gsem).wait()

  mu_r = jnp.full((_L,), mu_v[...][0], jnp.float32)
  inv_s = 1.0 / jnp.full((_L,), sig_v[...][0], jnp.float32)

  def init_emb(i, carry):
    j = i // (_TOKEN_DIM // _L)
    c = (i % (_TOKEN_DIM // _L)) * _L
    v = emb_v[j, pl.ds(c, _L)]
    for b in range(_NBUF):
      for k in range(_BATCH):
        bufs[b, k, j, pl.ds(c, _L)] = v
    return carry
  lax.fori_loop(0, _M * (_TOKEN_DIM // _L), init_emb, 0)

  def init_zero(i, carry):
    j = _M + i // (_TOKEN_DIM // _L)
    c = (i % (_TOKEN_DIM // _L)) * _L
    for b in range(_NBUF):
      for k in range(_BATCH):
        bufs[b, k, j, pl.ds(c, _L)] = zv
    return carry
  lax.fori_loop(0, (_MAX_LENGTH - _M) * (_TOKEN_DIM // _L), init_zero, 0)

  rows = [lax.iota(jnp.int32, _L) + g * _L for g in range(_M // _L)]
  c125 = jnp.full((_L,), 125, jnp.int32)
  c126 = jnp.full((_L,), 126, jnp.int32)
  c127 = jnp.full((_L,), 127, jnp.int32)
  zc = jnp.zeros((_L,), jnp.int32)
  oc = jnp.full((_L,), 1, jnp.int32)

  def round_body(r, carry):
    for b in range(_NBUF):
      tb = r * _NBUF * _BATCH + b * _BATCH

      @pl.when(r > 0)
      def _wait():
        pltpu.make_async_copy(bufs.at[b], out.at[pl.ds(t0, _BATCH)],
                              osem.at[b]).wait()

      for k in range(_BATCH):
        tt = tb + k
        tts = jnp.full((_L,), tt, jnp.int32)
        t0v = plsc.load_gather(tim_v, [tts, zc])
        t1v = plsc.load_gather(tim_v, [tts, oc])
        for g in range(_M // _L):
          v = plsc.load_gather(vals_v, [rows[g], tts])
          plsc.store_scatter(bufs.at[b, k], [rows[g], c125],
                             (v - mu_r) * inv_s)
          plsc.store_scatter(bufs.at[b, k], [rows[g], c126], t0v)
          plsc.store_scatter(bufs.at[b, k], [rows[g], c127], t1v)

      pltpu.make_async_copy(bufs.at[b], out.at[pl.ds(t0 + tb, _BATCH)],
                            osem.at[b]).start()
    return carry

  lax.fori_loop(0, _TPW // (_NBUF * _BATCH), round_body, 0)

  for b in range(_NBUF):
    pltpu.make_async_copy(bufs.at[b], out.at[pl.ds(t0, _BATCH)],
                          osem.at[b]).wait()


@jax.jit
def _tokenize(data, emb, node1, nbrs, mu1, sig1, tim):
  mesh = plsc.VectorSubcoreMesh(core_axis_name="c", subcore_axis_name="s",
                                num_cores=_NC, num_subcores=_NS)
  f = functools.partial(
      pl.kernel,
      out_type=jax.ShapeDtypeStruct((_N, _MAX_LENGTH, _TOKEN_DIM),
                                    jnp.float32),
      mesh=mesh,
      compiler_params=pltpu.CompilerParams(needs_layout_passes=False),
      scratch_types=[
          pltpu.VMEM((_M,), jnp.int32),
          pltpu.VMEM((_L,), jnp.int32),
          pltpu.VMEM((_M,), jnp.int32),
          pltpu.VMEM((_M, _TOKEN_DIM), jnp.float32),
          pltpu.VMEM((_M, _TPW), jnp.float32),
          pltpu.VMEM((_TPW, 2), jnp.float32),
          pltpu.VMEM((_L,), jnp.float32),
          pltpu.VMEM((_L,), jnp.float32),
          pltpu.VMEM((_NBUF, _BATCH, _MAX_LENGTH, _TOKEN_DIM),
                     jnp.float32),
          pltpu.SemaphoreType.DMA,
          pltpu.SemaphoreType.DMA((_NBUF,)),
      ],
  )(_tokenizer_body)
  return f(data, emb, node1, nbrs, mu1, sig1, tim)


def kernel(data, node, spatial_embedding, temporal_all, neighbors, zt_mu,
           zt_sigma):
  node1 = jnp.asarray(node, jnp.int32).reshape(1)
  return _tokenize(data, spatial_embedding, node1,
                   neighbors.astype(jnp.int32),
                   zt_mu.astype(jnp.float32), zt_sigma.astype(jnp.float32),
                   temporal_all)


# speedup vs baseline: 1.2579x; 1.0132x over previous
import functools

import jax
import jax.numpy as jnp
from jax import lax
from jax.experimental import pallas as pl
from jax.experimental.pallas import tpu as pltpu
from jax.experimental.pallas import tpu_sc as plsc

_NUM_NODES = 1000
_N = 4096
_D_SPATIAL = 125
_TOKEN_DIM = 128
_DEG = 31
_MAX_LENGTH = 64
_M = _DEG + 1

_NC = 2
_NS = 16
_NW = _NC * _NS
_TPW = _N // _NW
_L = 16
_NBUF = 4
_BATCH = 1


def _tokenizer_body(data, emb, node1, nbrs, mu1, sig1, tim, out,
                    ids_v, nd_v, nb_v, emb_v, vals_v, tim_v, mu_v, sig_v,
                    bufs, gsem, osem):
  wid = lax.axis_index("s") * _NC + lax.axis_index("c")
  t0 = wid * _TPW

  pltpu.sync_copy(node1, nd_v.at[pl.ds(0, 1)])
  pltpu.sync_copy(nbrs, nb_v.at[pl.ds(0, _DEG)])
  pltpu.sync_copy(mu1, mu_v.at[pl.ds(0, 1)])
  pltpu.sync_copy(sig1, sig_v.at[pl.ds(0, 1)])
  pltpu.sync_copy(tim.at[pl.ds(t0, _TPW)], tim_v)

  iota = lax.iota(jnp.int32, _L)
  plsc.store_scatter(ids_v, [iota], nd_v[...], mask=iota == 0)
  plsc.store_scatter(ids_v, [iota + 1], nb_v[pl.ds(0, _L)])
  plsc.store_scatter(ids_v, [iota + _L + 1], nb_v[pl.ds(_L, _L)],
                     mask=iota < _L - 1)

  zv = jnp.zeros((_L,), jnp.float32)

  def init_tail(j, carry):
    emb_v[j, pl.ds(_TOKEN_DIM - _L, _L)] = zv
    return carry
  lax.fori_loop(0, _M, init_tail, 0)

  idv = [ids_v[pl.ds(g * _L, _L)] for g in range(_M // _L)]
  for j in range(_M):
    idj = idv[j // _L][j % _L]
    pltpu.make_async_copy(data.at[idj, pl.ds(t0, _TPW)],
                          vals_v.at[j], gsem).start()
    pltpu.make_async_copy(emb.at[idj], emb_v.at[j, pl.ds(0, _D_SPATIAL)],
                          gsem).start()

  def init_zero(i, carry):
    j = _M + i // (_TOKEN_DIM // _L)
    c = (i % (_TOKEN_DIM // _L)) * _L
    for b in range(_NBUF):
      for k in range(_BATCH):
        bufs[b, k, j, pl.ds(c, _L)] = zv
    return carry
  lax.fori_loop(0, (_MAX_LENGTH - _M) * (_TOKEN_DIM // _L), init_zero, 0)

  for j in range(_M):
    pltpu.make_async_copy(data.at[0, pl.ds(t0, _TPW)],
                          vals_v.at[j], gsem).wait()
    pltpu.make_async_copy(emb.at[0], emb_v.at[j, pl.ds(0, _D_SPATIAL)],
                          gsem).wait()

  mu_r = jnp.full((_L,), mu_v[...][0], jnp.float32)
  inv_s = 1.0 / jnp.full((_L,), sig_v[...][0], jnp.float32)

  def init_emb(i, carry):
    j = i // (_TOKEN_DIM // _L)
    c = (i % (_TOKEN_DIM // _L)) * _L
    v = emb_v[j, pl.ds(c, _L)]
    for b in range(_NBUF):
      for k in range(_BATCH):
        bufs[b, k, j, pl.ds(c, _L)] = v
    return carry
  lax.fori_loop(0, _M * (_TOKEN_DIM // _L), init_emb, 0)

  rows = [lax.iota(jnp.int32, _L) + g * _L for g in range(_M // _L)]
  c125 = jnp.full((_L,), 125, jnp.int32)
  c126 = jnp.full((_L,), 126, jnp.int32)
  c127 = jnp.full((_L,), 127, jnp.int32)
  zc = jnp.zeros((_L,), jnp.int32)
  oc = jnp.full((_L,), 1, jnp.int32)

  def round_body(r, carry):
    for b in range(_NBUF):
      tb = r * _NBUF * _BATCH + b * _BATCH

      @pl.when(r > 0)
      def _wait():
        pltpu.make_async_copy(bufs.at[b], out.at[pl.ds(t0, _BATCH)],
                              osem.at[b]).wait()

      for k in range(_BATCH):
        tt = tb + k
        tts = jnp.full((_L,), tt, jnp.int32)
        t0v = plsc.load_gather(tim_v, [tts, zc])
        t1v = plsc.load_gather(tim_v, [tts, oc])
        for g in range(_M // _L):
          v = plsc.load_gather(vals_v, [rows[g], tts])
          plsc.store_scatter(bufs.at[b, k], [rows[g], c125],
                             (v - mu_r) * inv_s)
          plsc.store_scatter(bufs.at[b, k], [rows[g], c126], t0v)
          plsc.store_scatter(bufs.at[b, k], [rows[g], c127], t1v)

      pltpu.make_async_copy(bufs.at[b], out.at[pl.ds(t0 + tb, _BATCH)],
                            osem.at[b]).start()
    return carry

  lax.fori_loop(0, _TPW // (_NBUF * _BATCH), round_body, 0)

  for b in range(_NBUF):
    pltpu.make_async_copy(bufs.at[b], out.at[pl.ds(t0, _BATCH)],
                          osem.at[b]).wait()


@jax.jit
def _tokenize(data, emb, node1, nbrs, mu1, sig1, tim):
  mesh = plsc.VectorSubcoreMesh(core_axis_name="c", subcore_axis_name="s",
                                num_cores=_NC, num_subcores=_NS)
  f = functools.partial(
      pl.kernel,
      out_type=jax.ShapeDtypeStruct((_N, _MAX_LENGTH, _TOKEN_DIM),
                                    jnp.float32),
      mesh=mesh,
      compiler_params=pltpu.CompilerParams(needs_layout_passes=False),
      scratch_types=[
          pltpu.VMEM((_M,), jnp.int32),
          pltpu.VMEM((_L,), jnp.int32),
          pltpu.VMEM((_M,), jnp.int32),
          pltpu.VMEM((_M, _TOKEN_DIM), jnp.float32),
          pltpu.VMEM((_M, _TPW), jnp.float32),
          pltpu.VMEM((_TPW, 2), jnp.float32),
          pltpu.VMEM((_L,), jnp.float32),
          pltpu.VMEM((_L,), jnp.float32),
          pltpu.VMEM((_NBUF, _BATCH, _MAX_LENGTH, _TOKEN_DIM),
                     jnp.float32),
          pltpu.SemaphoreType.DMA,
          pltpu.SemaphoreType.DMA((_NBUF,)),
      ],
  )(_tokenizer_body)
  return f(data, emb, node1, nbrs, mu1, sig1, tim)


def kernel(data, node, spatial_embedding, temporal_all, neighbors, zt_mu,
           zt_sigma):
  node1 = jnp.asarray(node, jnp.int32).reshape(1)
  return _tokenize(data, spatial_embedding, node1, neighbors,
                   zt_mu, zt_sigma, temporal_all)
